# baseline (speedup 1.0000x reference)
import functools

import jax
import jax.numpy as jnp
from jax import lax
from jax.experimental import pallas as pl
from jax.experimental.pallas import tpu as pltpu
from jax.experimental.pallas import tpu_sc as plsc

NC = 2
NS = 16
NW = NC * NS
L = 16

F = 128


def _mesh():
    return plsc.VectorSubcoreMesh(core_axis_name="c", subcore_axis_name="s")


def _make_agg(n_pad, e, ch):
    ept = e // NW
    rpt = n_pad // NS
    n_chunks = ept // ch

    assert n_chunks % 2 == 0
    scratch = (
        pltpu.VMEM_SHARED((n_pad, F), jnp.float32),
        pltpu.VMEM((ch, F), jnp.float32),
        pltpu.VMEM((ch, F), jnp.float32),
        pltpu.VMEM((ch,), jnp.int32),
        pltpu.VMEM((ch,), jnp.int32),
        pltpu.VMEM((ch,), jnp.int32),
        pltpu.VMEM((ch,), jnp.int32),
        pltpu.SemaphoreType.DMA,
        pltpu.SemaphoreType.DMA,
    )

    def body(feat_hbm, src_hbm, dst_hbm, agg_out, table, msga, msgb,
             sbufa, sbufb, dbufa, dbufb, sema, semb):
        c = lax.axis_index("c")
        s = lax.axis_index("s")
        wid = c * NS + s

        def zmsg(i, _):
            for j in range(F // L):
                msga[i, pl.ds(j * L, L)] = jnp.zeros((L,), jnp.float32)
            return 0
        lax.fori_loop(0, ch, zmsg, 0)
        off = 0
        while off < rpt:
            step = min(ch, rpt - off)
            pltpu.sync_copy(msga.at[pl.ds(0, step)],
                            table.at[pl.ds(s * rpt + off, step)])
            off += step
        plsc.subcore_barrier()

        ebase = wid * ept

        def pair(k2, _):
            base_a = ebase + (2 * k2) * ch
            base_b = base_a + ch
            pltpu.sync_copy(src_hbm.at[pl.ds(base_a, ch)], sbufa)
            pltpu.sync_copy(src_hbm.at[pl.ds(base_b, ch)], sbufb)
            pltpu.sync_copy(dst_hbm.at[pl.ds(base_a, ch)], dbufa)
            pltpu.sync_copy(dst_hbm.at[pl.ds(base_b, ch)], dbufb)
            cpa = pltpu.async_copy(feat_hbm.at[sbufa], msga, sema)
            cpb = pltpu.async_copy(feat_hbm.at[sbufb], msgb, semb)
            cpa.wait()
            pltpu.sync_copy(msga, table.at[dbufa], add=True)
            cpb.wait()
            pltpu.sync_copy(msgb, table.at[dbufb], add=True)
            return 0
        lax.fori_loop(0, n_chunks // 2, pair, 0)
        plsc.subcore_barrier()

        obase = c * n_pad + s * rpt
        off = 0
        while off < rpt:
            step = min(ch, rpt - off)
            pltpu.sync_copy(table.at[pl.ds(s * rpt + off, step)],
                            agg_out.at[pl.ds(obase + off, step)])
            off += step

    return pl.kernel(body,
                     out_type=jax.ShapeDtypeStruct((NC * n_pad, F),
                                                   jnp.float32),
                     mesh=_mesh(), scratch_types=scratch)


def _make_deg(n_pad, e, ch):
    ept = e // NW
    rpt = n_pad // NS
    n_chunks = ept // ch

    scratch = (
        pltpu.VMEM_SHARED((n_pad, F), jnp.float32),
        pltpu.VMEM((ch, F), jnp.float32),
        pltpu.VMEM((ch,), jnp.int32),
    )

    def body(dst_hbm, deg_out, dtable, ones, dbuf):
        c = lax.axis_index("c")
        s = lax.axis_index("s")
        wid = c * NS + s

        def zones(i, _):
            for j in range(F // L):
                ones[i, pl.ds(j * L, L)] = jnp.zeros((L,), jnp.float32)
            return 0
        lax.fori_loop(0, ch, zones, 0)
        off = 0
        while off < rpt:
            step = min(ch, rpt - off)
            pltpu.sync_copy(ones.at[pl.ds(0, step)],
                            dtable.at[pl.ds(s * rpt + off, step)])
            off += step

        def fones(i, _):
            for j in range(F // L):
                ones[i, pl.ds(j * L, L)] = jnp.ones((L,), jnp.float32)
            return 0
        lax.fori_loop(0, ch, fones, 0)
        plsc.subcore_barrier()

        ebase = wid * ept

        def chunk(k, _):
            base = ebase + k * ch
            pltpu.sync_copy(dst_hbm.at[pl.ds(base, ch)], dbuf)
            pltpu.sync_copy(ones, dtable.at[dbuf], add=True)
            return 0
        lax.fori_loop(0, n_chunks, chunk, 0)
        plsc.subcore_barrier()

        obase = c * n_pad + s * rpt
        off = 0
        while off < rpt:
            step = min(ch, rpt - off)
            pltpu.sync_copy(dtable.at[pl.ds(s * rpt + off, step)],
                            deg_out.at[pl.ds(obase + off, step)])
            off += step

    return pl.kernel(body,
                     out_type=jax.ShapeDtypeStruct((NC * n_pad, F),
                                                   jnp.float32),
                     mesh=_mesh(), scratch_types=scratch)


def _make_classifier(e, ch):
    ept = e // NW
    n_chunks = ept // ch

    assert n_chunks % 2 == 0
    scratch = (
        pltpu.VMEM((ch, F), jnp.float32),
        pltpu.VMEM((ch, F), jnp.float32),
        pltpu.VMEM((ch, F), jnp.float32),
        pltpu.VMEM((ch, F), jnp.float32),
        pltpu.VMEM((ch, F), jnp.float32),
        pltpu.VMEM((ch, F), jnp.float32),
        pltpu.VMEM((ch,), jnp.int32),
        pltpu.VMEM((ch,), jnp.int32),
        pltpu.VMEM((ch,), jnp.int32),
        pltpu.VMEM((ch,), jnp.int32),
        pltpu.VMEM((ch, L), jnp.float32),
        pltpu.VMEM((ch, L), jnp.float32),
        pltpu.VMEM((F,), jnp.float32),
        pltpu.VMEM((L,), jnp.float32),
        pltpu.SemaphoreType.DMA,
        pltpu.SemaphoreType.DMA,
        pltpu.SemaphoreType.DMA,
        pltpu.SemaphoreType.DMA,
        pltpu.SemaphoreType.DMA,
        pltpu.SemaphoreType.DMA,
    )

    def body(p_hbm, q_hbm, t_hbm, src_hbm, dst_hbm, w_hbm, a0_hbm, out_hbm,
             bufpa, bufpb, bufqa, bufqb, bufta, buftb,
             sbufa, sbufb, dbufa, dbufb, obufa, obufb, wbuf, a0buf, *sems):
        c = lax.axis_index("c")
        s = lax.axis_index("s")
        wid = c * NS + s
        pltpu.sync_copy(w_hbm, wbuf)
        pltpu.sync_copy(a0_hbm, a0buf)
        wvecs = [wbuf[pl.ds(j * L, L)] for j in range(F // L)]
        acc0 = a0buf[...]
        ebase = wid * ept
        bufs = ((bufpa, bufqa, bufta, sbufa, dbufa, obufa),
                (bufpb, bufqb, buftb, sbufb, dbufb, obufb))

        def compute(side, base):
            bp, bq, bt, _, _, ob = bufs[side]

            def edge(i, _):
                acc = acc0
                for j in range(F // L):
                    a = bp[i, pl.ds(j * L, L)]
                    b = bq[i, pl.ds(j * L, L)]
                    t = bt[i, pl.ds(j * L, L)]
                    z = jnp.maximum(a + b + t, 0.0)
                    acc = acc + z * wvecs[j]
                ob[i, pl.ds(0, L)] = acc
                return 0
            lax.fori_loop(0, ch, edge, 0)
            pltpu.sync_copy(ob, out_hbm.at[pl.ds(base, ch)])

        def pair(k2, _):
            base_a = ebase + (2 * k2) * ch
            base_b = base_a + ch
            cps = []
            for side, base in ((0, base_a), (1, base_b)):
                bp, bq, bt, sb, db, _ = bufs[side]
                pltpu.sync_copy(src_hbm.at[pl.ds(base, ch)], sb)
                pltpu.sync_copy(dst_hbm.at[pl.ds(base, ch)], db)
                cps.append((
                    pltpu.async_copy(p_hbm.at[sb], bp, sems[3 * side]),
                    pltpu.async_copy(q_hbm.at[db], bq, sems[3 * side + 1]),
                    pltpu.async_copy(t_hbm.at[pl.ds(base, ch)], bt,
                                     sems[3 * side + 2]),
                ))
            for side, base in ((0, base_a), (1, base_b)):
                for cp in cps[side]:
                    cp.wait()
                compute(side, base)
            return 0
        lax.fori_loop(0, n_chunks // 2, pair, 0)

    return pl.kernel(body, out_type=jax.ShapeDtypeStruct((e, L), jnp.float32),
                     mesh=_mesh(), scratch_types=scratch)


def _tc_layer(aggp, degp, feat, Wl, bl, Wr, relu, blk=512):
    n_pad = feat.shape[0]

    def body(a_ref, d_ref, f_ref, wl_ref, bl_ref, wr_ref, o_ref):
        agg = a_ref[0] + a_ref[1]
        deg = d_ref[0, :, 0:1] + d_ref[1, :, 0:1]
        mean = agg / jnp.maximum(deg, 1.0)
        h = (jnp.dot(mean, wl_ref[...], preferred_element_type=jnp.float32)
             + bl_ref[...]
             + jnp.dot(f_ref[...], wr_ref[...],
                       preferred_element_type=jnp.float32))
        if relu:
            h = jnp.maximum(h, 0.0)
        o_ref[...] = h

    return pl.pallas_call(
        body,
        grid=(n_pad // blk,),
        in_specs=[
            pl.BlockSpec((NC, blk, F), lambda i: (0, i, 0)),
            pl.BlockSpec((NC, blk, F), lambda i: (0, i, 0)),
            pl.BlockSpec((blk, F), lambda i: (i, 0)),
            pl.BlockSpec((F, F), lambda i: (0, 0)),
            pl.BlockSpec((1, F), lambda i: (0, 0)),
            pl.BlockSpec((F, F), lambda i: (0, 0)),
        ],
        out_specs=pl.BlockSpec((blk, F), lambda i: (i, 0)),
        out_shape=jax.ShapeDtypeStruct((n_pad, F), jnp.float32),
    )(aggp.reshape(NC, n_pad, F), degp.reshape(NC, n_pad, F), feat,
      Wl, bl.reshape(1, F), Wr)


def _tc_layer2_pq(aggp, degp, feat, Wl, bl, Wr, Wa, Wb, blk=512):
    n_pad = feat.shape[0]

    def body(a_ref, d_ref, f_ref, wl_ref, bl_ref, wr_ref, wa_ref, wb_ref,
             p_ref, q_ref):
        agg = a_ref[0] + a_ref[1]
        deg = d_ref[0, :, 0:1] + d_ref[1, :, 0:1]
        mean = agg / jnp.maximum(deg, 1.0)
        h2 = (jnp.dot(mean, wl_ref[...], preferred_element_type=jnp.float32)
              + bl_ref[...]
              + jnp.dot(f_ref[...], wr_ref[...],
                        preferred_element_type=jnp.float32))
        p_ref[...] = jnp.dot(h2, wa_ref[...], preferred_element_type=jnp.float32)
        q_ref[...] = jnp.dot(h2, wb_ref[...], preferred_element_type=jnp.float32)

    return pl.pallas_call(
        body,
        grid=(n_pad // blk,),
        in_specs=[
            pl.BlockSpec((NC, blk, F), lambda i: (0, i, 0)),
            pl.BlockSpec((NC, blk, F), lambda i: (0, i, 0)),
            pl.BlockSpec((blk, F), lambda i: (i, 0)),
            pl.BlockSpec((F, F), lambda i: (0, 0)),
            pl.BlockSpec((1, F), lambda i: (0, 0)),
            pl.BlockSpec((F, F), lambda i: (0, 0)),
            pl.BlockSpec((F, F), lambda i: (0, 0)),
            pl.BlockSpec((F, F), lambda i: (0, 0)),
        ],
        out_specs=[
            pl.BlockSpec((blk, F), lambda i: (i, 0)),
            pl.BlockSpec((blk, F), lambda i: (i, 0)),
        ],
        out_shape=[
            jax.ShapeDtypeStruct((n_pad, F), jnp.float32),
            jax.ShapeDtypeStruct((n_pad, F), jnp.float32),
        ],
    )(aggp.reshape(NC, n_pad, F), degp.reshape(NC, n_pad, F), feat,
      Wl, bl.reshape(1, F), Wr, Wa, Wb)


def _tc_rowsum(part, blk=3200):
    e = part.shape[0]

    def body(z_ref, o_ref):
        o_ref[...] = jnp.sum(z_ref[...], axis=1, keepdims=True)

    return pl.pallas_call(
        body,
        grid=(e // blk,),
        in_specs=[pl.BlockSpec((blk, L), lambda i: (i, 0))],
        out_specs=pl.BlockSpec((blk, 1), lambda i: (i, 0)),
        out_shape=jax.ShapeDtypeStruct((e, 1), jnp.float32),
    )(part)


def _tc_edge_term(edge_attr, Wc, bc, blk=2048):
    e, de = edge_attr.shape

    def body(ea_ref, wc_ref, bc_ref, t_ref):
        t_ref[...] = (jnp.dot(ea_ref[...], wc_ref[...],
                              preferred_element_type=jnp.float32)
                      + bc_ref[...])

    return pl.pallas_call(
        body,
        grid=(e // blk,),
        in_specs=[
            pl.BlockSpec((blk, de), lambda i: (i, 0)),
            pl.BlockSpec((de, F), lambda i: (0, 0)),
            pl.BlockSpec((1, F), lambda i: (0, 0)),
        ],
        out_specs=pl.BlockSpec((blk, F), lambda i: (i, 0)),
        out_shape=jax.ShapeDtypeStruct((e, F), jnp.float32),
    )(edge_attr, Wc, bc.reshape(1, F))


def kernel(x, edge_index, edge_attr, Wl1, bl1, Wr1, Wl2, bl2, Wr2,
           Wc1, bc1, Wc2, bc2):
    n = x.shape[0]
    e = edge_index.shape[1]
    n_pad = ((n + 511) // 512) * 512
    ch = 96
    ept_pad = ((e // NW + 2 * ch - 1) // (2 * ch)) * (2 * ch)
    e_pad = ept_pad * NW
    src = edge_index[0]
    dst = edge_index[1]

    x_pad = jnp.pad(x, ((0, n_pad - n), (0, 0)))
    src_p = jnp.pad(src, (0, e_pad - e))
    dst_p = jnp.pad(dst, (0, e_pad - e), constant_values=n_pad - 1)
    ea_p = jnp.pad(edge_attr, ((0, e_pad - e), (0, 0)))

    degp = _make_deg(n_pad, e_pad, ch)(dst_p)
    agg1p = _make_agg(n_pad, e_pad, ch)(x_pad, src_p, dst_p)
    h1 = _tc_layer(agg1p, degp, x_pad, Wl1, bl1, Wr1, relu=True)
    agg2p = _make_agg(n_pad, e_pad, ch)(h1, src_p, dst_p)
    p_tab, q_tab = _tc_layer2_pq(agg2p, degp, h1, Wl2, bl2, Wr2,
                                 Wc1[0:F], Wc1[F:2 * F])
    t_term = _tc_edge_term(ea_p, Wc1[2 * F:], bc1)

    acc0 = jnp.zeros((L,), jnp.float32).at[0].set(bc2[0])
    partials = _make_classifier(e_pad, ch)(p_tab, q_tab, t_term, src_p, dst_p,
                                           Wc2.reshape(F), acc0)
    return _tc_rowsum(partials[:e])


# speedup vs baseline: 1.4874x; 1.4874x over previous
import functools

import jax
import jax.numpy as jnp
from jax import lax
from jax.experimental import pallas as pl
from jax.experimental.pallas import tpu as pltpu
from jax.experimental.pallas import tpu_sc as plsc

NC = 2
NS = 16
NW = NC * NS
L = 16

F = 128


def _mesh():
    return plsc.VectorSubcoreMesh(core_axis_name="c", subcore_axis_name="s")


def _make_agg(n_pad, e, ch):
    ept = e // NW
    rpt = n_pad // NS
    n_chunks = ept // ch

    scratch = (
        pltpu.VMEM_SHARED((n_pad, F), jnp.float32),
        pltpu.VMEM((ch, F), jnp.float32),
        pltpu.VMEM((ch,), jnp.int32),
        pltpu.VMEM((ch,), jnp.int32),
        pltpu.SemaphoreType.DMA,
    )

    def body(feat_hbm, src_hbm, dst_hbm, agg_out, table, msg, sbuf, dbuf, sem):
        c = lax.axis_index("c")
        s = lax.axis_index("s")
        wid = c * NS + s

        def zmsg(i, _):
            for j in range(F // L):
                msg[i, pl.ds(j * L, L)] = jnp.zeros((L,), jnp.float32)
            return 0
        lax.fori_loop(0, ch, zmsg, 0)
        off = 0
        while off < rpt:
            step = min(ch, rpt - off)
            pltpu.sync_copy(msg.at[pl.ds(0, step)],
                            table.at[pl.ds(s * rpt + off, step)])
            off += step
        plsc.subcore_barrier()

        ebase = wid * ept

        def chunk(k, _):
            base = ebase + k * ch
            pltpu.sync_copy(src_hbm.at[pl.ds(base, ch)], sbuf)
            pltpu.sync_copy(dst_hbm.at[pl.ds(base, ch)], dbuf)
            pltpu.async_copy(feat_hbm.at[sbuf], msg, sem).wait()
            pltpu.sync_copy(msg, table.at[dbuf], add=True)
            return 0
        lax.fori_loop(0, n_chunks, chunk, 0)
        plsc.subcore_barrier()

        obase = c * n_pad + s * rpt
        off = 0
        while off < rpt:
            step = min(ch, rpt - off)
            pltpu.sync_copy(table.at[pl.ds(s * rpt + off, step)],
                            agg_out.at[pl.ds(obase + off, step)])
            off += step

    return pl.kernel(body,
                     out_type=jax.ShapeDtypeStruct((NC * n_pad, F),
                                                   jnp.float32),
                     mesh=_mesh(), scratch_types=scratch)


def _make_deg(n_pad, e, ch):
    ept = e // NW
    rpt = n_pad // NS
    n_chunks = ept // ch

    scratch = (
        pltpu.VMEM_SHARED((n_pad, F), jnp.float32),
        pltpu.VMEM((ch, F), jnp.float32),
        pltpu.VMEM((ch,), jnp.int32),
    )

    def body(dst_hbm, deg_out, dtable, ones, dbuf):
        c = lax.axis_index("c")
        s = lax.axis_index("s")
        wid = c * NS + s

        def zones(i, _):
            for j in range(F // L):
                ones[i, pl.ds(j * L, L)] = jnp.zeros((L,), jnp.float32)
            return 0
        lax.fori_loop(0, ch, zones, 0)
        off = 0
        while off < rpt:
            step = min(ch, rpt - off)
            pltpu.sync_copy(ones.at[pl.ds(0, step)],
                            dtable.at[pl.ds(s * rpt + off, step)])
            off += step

        def fones(i, _):
            for j in range(F // L):
                ones[i, pl.ds(j * L, L)] = jnp.ones((L,), jnp.float32)
            return 0
        lax.fori_loop(0, ch, fones, 0)
        plsc.subcore_barrier()

        ebase = wid * ept

        def chunk(k, _):
            base = ebase + k * ch
            pltpu.sync_copy(dst_hbm.at[pl.ds(base, ch)], dbuf)
            pltpu.sync_copy(ones, dtable.at[dbuf], add=True)
            return 0
        lax.fori_loop(0, n_chunks, chunk, 0)
        plsc.subcore_barrier()

        obase = c * n_pad + s * rpt
        off = 0
        while off < rpt:
            step = min(ch, rpt - off)
            pltpu.sync_copy(dtable.at[pl.ds(s * rpt + off, step)],
                            deg_out.at[pl.ds(obase + off, step)])
            off += step

    return pl.kernel(body,
                     out_type=jax.ShapeDtypeStruct((NC * n_pad, F),
                                                   jnp.float32),
                     mesh=_mesh(), scratch_types=scratch)


def _make_classifier(e, ch):
    ept = e // NW
    n_chunks = ept // ch

    scratch = (
        pltpu.VMEM((ch, F), jnp.float32),
        pltpu.VMEM((ch, F), jnp.float32),
        pltpu.VMEM((ch, F), jnp.float32),
        pltpu.VMEM((ch,), jnp.int32),
        pltpu.VMEM((ch,), jnp.int32),
        pltpu.VMEM((ch, L), jnp.float32),
        pltpu.VMEM((F,), jnp.float32),
        pltpu.VMEM((L,), jnp.float32),
        pltpu.SemaphoreType.DMA,
        pltpu.SemaphoreType.DMA,
    )

    def body(p_hbm, q_hbm, t_hbm, src_hbm, dst_hbm, w_hbm, a0_hbm, out_hbm,
             bufa, bufb, buft, sbuf, dbuf, obuf, wbuf, a0buf, sem1, sem2):
        c = lax.axis_index("c")
        s = lax.axis_index("s")
        wid = c * NS + s
        pltpu.sync_copy(w_hbm, wbuf)
        pltpu.sync_copy(a0_hbm, a0buf)
        wvecs = [wbuf[pl.ds(j * L, L)] for j in range(F // L)]
        acc0 = a0buf[...]
        ebase = wid * ept

        def chunk(k, _):
            base = ebase + k * ch
            pltpu.sync_copy(src_hbm.at[pl.ds(base, ch)], sbuf)
            pltpu.sync_copy(dst_hbm.at[pl.ds(base, ch)], dbuf)
            cp1 = pltpu.async_copy(p_hbm.at[sbuf], bufa, sem1)
            cp2 = pltpu.async_copy(q_hbm.at[dbuf], bufb, sem2)
            pltpu.sync_copy(t_hbm.at[pl.ds(base, ch)], buft)
            cp1.wait()
            cp2.wait()

            def edge(i, _):
                acc = acc0
                for j in range(F // L):
                    a = bufa[i, pl.ds(j * L, L)]
                    b = bufb[i, pl.ds(j * L, L)]
                    t = buft[i, pl.ds(j * L, L)]
                    z = jnp.maximum(a + b + t, 0.0)
                    acc = acc + z * wvecs[j]
                obuf[i, pl.ds(0, L)] = acc
                return 0
            lax.fori_loop(0, ch, edge, 0)
            pltpu.sync_copy(obuf, out_hbm.at[pl.ds(base, ch)])
            return 0
        lax.fori_loop(0, n_chunks, chunk, 0)

    return pl.kernel(body, out_type=jax.ShapeDtypeStruct((e, L), jnp.float32),
                     mesh=_mesh(), scratch_types=scratch)


def _tc_layer(aggp, degp, feat, Wl, bl, Wr, relu, blk=512):
    n_pad = feat.shape[0]

    def body(a_ref, d_ref, f_ref, wl_ref, bl_ref, wr_ref, o_ref):
        agg = a_ref[0] + a_ref[1]
        deg = d_ref[0, :, 0:1] + d_ref[1, :, 0:1]
        mean = agg / jnp.maximum(deg, 1.0)
        h = (jnp.dot(mean, wl_ref[...], preferred_element_type=jnp.float32)
             + bl_ref[...]
             + jnp.dot(f_ref[...], wr_ref[...],
                       preferred_element_type=jnp.float32))
        if relu:
            h = jnp.maximum(h, 0.0)
        o_ref[...] = h

    return pl.pallas_call(
        body,
        grid=(n_pad // blk,),
        in_specs=[
            pl.BlockSpec((NC, blk, F), lambda i: (0, i, 0)),
            pl.BlockSpec((NC, blk, F), lambda i: (0, i, 0)),
            pl.BlockSpec((blk, F), lambda i: (i, 0)),
            pl.BlockSpec((F, F), lambda i: (0, 0)),
            pl.BlockSpec((1, F), lambda i: (0, 0)),
            pl.BlockSpec((F, F), lambda i: (0, 0)),
        ],
        out_specs=pl.BlockSpec((blk, F), lambda i: (i, 0)),
        out_shape=jax.ShapeDtypeStruct((n_pad, F), jnp.float32),
    )(aggp.reshape(NC, n_pad, F), degp.reshape(NC, n_pad, F), feat,
      Wl, bl.reshape(1, F), Wr)


def _tc_layer2_pq(aggp, degp, feat, Wl, bl, Wr, Wa, Wb, blk=512):
    n_pad = feat.shape[0]

    def body(a_ref, d_ref, f_ref, wl_ref, bl_ref, wr_ref, wa_ref, wb_ref,
             p_ref, q_ref):
        agg = a_ref[0] + a_ref[1]
        deg = d_ref[0, :, 0:1] + d_ref[1, :, 0:1]
        mean = agg / jnp.maximum(deg, 1.0)
        h2 = (jnp.dot(mean, wl_ref[...], preferred_element_type=jnp.float32)
              + bl_ref[...]
              + jnp.dot(f_ref[...], wr_ref[...],
                        preferred_element_type=jnp.float32))
        p_ref[...] = jnp.dot(h2, wa_ref[...], preferred_element_type=jnp.float32)
        q_ref[...] = jnp.dot(h2, wb_ref[...], preferred_element_type=jnp.float32)

    return pl.pallas_call(
        body,
        grid=(n_pad // blk,),
        in_specs=[
            pl.BlockSpec((NC, blk, F), lambda i: (0, i, 0)),
            pl.BlockSpec((NC, blk, F), lambda i: (0, i, 0)),
            pl.BlockSpec((blk, F), lambda i: (i, 0)),
            pl.BlockSpec((F, F), lambda i: (0, 0)),
            pl.BlockSpec((1, F), lambda i: (0, 0)),
            pl.BlockSpec((F, F), lambda i: (0, 0)),
            pl.BlockSpec((F, F), lambda i: (0, 0)),
            pl.BlockSpec((F, F), lambda i: (0, 0)),
        ],
        out_specs=[
            pl.BlockSpec((blk, F), lambda i: (i, 0)),
            pl.BlockSpec((blk, F), lambda i: (i, 0)),
        ],
        out_shape=[
            jax.ShapeDtypeStruct((n_pad, F), jnp.float32),
            jax.ShapeDtypeStruct((n_pad, F), jnp.float32),
        ],
    )(aggp.reshape(NC, n_pad, F), degp.reshape(NC, n_pad, F), feat,
      Wl, bl.reshape(1, F), Wr, Wa, Wb)


def _tc_rowsum(part, blk=3200):
    e = part.shape[0]

    def body(z_ref, o_ref):
        o_ref[...] = jnp.sum(z_ref[...], axis=1, keepdims=True)

    return pl.pallas_call(
        body,
        grid=(e // blk,),
        in_specs=[pl.BlockSpec((blk, L), lambda i: (i, 0))],
        out_specs=pl.BlockSpec((blk, 1), lambda i: (i, 0)),
        out_shape=jax.ShapeDtypeStruct((e, 1), jnp.float32),
    )(part)


def _tc_edge_term(edge_attr, Wc, bc, blk=3200):
    e, de = edge_attr.shape

    def body(ea_ref, wc_ref, bc_ref, t_ref):
        t_ref[...] = (jnp.dot(ea_ref[...], wc_ref[...],
                              preferred_element_type=jnp.float32)
                      + bc_ref[...])

    return pl.pallas_call(
        body,
        grid=(e // blk,),
        in_specs=[
            pl.BlockSpec((blk, de), lambda i: (i, 0)),
            pl.BlockSpec((de, F), lambda i: (0, 0)),
            pl.BlockSpec((1, F), lambda i: (0, 0)),
        ],
        out_specs=pl.BlockSpec((blk, F), lambda i: (i, 0)),
        out_shape=jax.ShapeDtypeStruct((e, F), jnp.float32),
    )(edge_attr, Wc, bc.reshape(1, F))


def kernel(x, edge_index, edge_attr, Wl1, bl1, Wr1, Wl2, bl2, Wr2,
           Wc1, bc1, Wc2, bc2):
    n = x.shape[0]
    e = edge_index.shape[1]
    n_pad = ((n + 511) // 512) * 512
    src = edge_index[0]
    dst = edge_index[1]

    x_pad = jnp.pad(x, ((0, n_pad - n), (0, 0)))

    degp = _make_deg(n_pad, e, 80)(dst)
    agg1p = _make_agg(n_pad, e, 80)(x_pad, src, dst)
    h1 = _tc_layer(agg1p, degp, x_pad, Wl1, bl1, Wr1, relu=True)
    agg2p = _make_agg(n_pad, e, 80)(h1, src, dst)
    p_tab, q_tab = _tc_layer2_pq(agg2p, degp, h1, Wl2, bl2, Wr2,
                                 Wc1[0:F], Wc1[F:2 * F])
    t_term = _tc_edge_term(edge_attr, Wc1[2 * F:], bc1)

    acc0 = jnp.zeros((L,), jnp.float32).at[0].set(bc2[0])
    partials = _make_classifier(e, 80)(p_tab, q_tab, t_term, src, dst,
                                       Wc2.reshape(F), acc0)
    return _tc_rowsum(partials)
